# baseline (speedup 1.0000x reference)
import jax
import jax.numpy as jnp
from jax import lax
from jax.experimental import pallas as pl
from jax.experimental.pallas import tpu as pltpu
from jax.experimental.pallas import tpu_sc as plsc

D = 32
NC = 2
NS = 16
NW = NC * NS
SUB = 128
SPC = 4
CH = SUB * SPC
GRP = 16


def _mf_body(ut, st, P, Q, Pb, Qb, out,
             uid_v, sid_v,
             p0, q0, pb0, qb0, o0,
             p1, q1, pb1, qb1, o1,
             g0, g1, os0, os1):
    L = ut.shape[0]
    nch = L // SPC
    wid = lax.axis_index("s") * NC + lax.axis_index("c")
    iota = lax.iota(jnp.int32, GRP)
    bufs = ((p0, q0, pb0, qb0, o0, g0, os0),
            (p1, q1, pb1, qb1, o1, g1, os1))

    pltpu.sync_copy(ut.at[pl.ds(0, L), pl.ds(wid * SUB, SUB)], uid_v)
    pltpu.sync_copy(st.at[pl.ds(0, L), pl.ds(wid * SUB, SUB)], sid_v)

    def fire(cc, k):
        p_v, q_v, pb_v, qb_v, _, gsem, _ = bufs[k]
        for j in range(SPC):
            row = cc * SPC + j
            dst = pl.ds(j * SUB, SUB)
            pltpu.async_copy(P.at[uid_v.at[row]], p_v.at[dst], gsem)
            pltpu.async_copy(Q.at[sid_v.at[row]], q_v.at[dst], gsem)
            pltpu.async_copy(Pb.at[uid_v.at[row]], pb_v.at[dst], gsem)
            pltpu.async_copy(Qb.at[sid_v.at[row]], qb_v.at[dst], gsem)

    def drain_gathers(cc, k):
        p_v, q_v, pb_v, qb_v, _, gsem, _ = bufs[k]
        for j in range(SPC):
            row = cc * SPC + j
            dst = pl.ds(j * SUB, SUB)
            pltpu.make_async_copy(P.at[uid_v.at[row]], p_v.at[dst], gsem).wait()
            pltpu.make_async_copy(Q.at[sid_v.at[row]], q_v.at[dst], gsem).wait()
            pltpu.make_async_copy(Pb.at[uid_v.at[row]], pb_v.at[dst],
                                  gsem).wait()
            pltpu.make_async_copy(Qb.at[sid_v.at[row]], qb_v.at[dst],
                                  gsem).wait()

    def drain_out(k):
        _, _, _, _, o_v, _, osem = bufs[k]
        pltpu.make_async_copy(
            o_v, out.at[pl.ds(0, SPC), pl.ds(0, SUB)], osem).wait()

    def compute(cc, k):
        p_v, q_v, pb_v, qb_v, o_v, _, _ = bufs[k]

        def grp_body(g, carry):
            hs = []
            for r in range(GRP):
                i = g * GRP + r
                p1, p2 = plsc.unpack(p_v[i, pl.ds(0, D)],
                                     format=plsc.PackFormat.INTERLEAVED)
                q1, q2 = plsc.unpack(q_v[i, pl.ds(0, D)],
                                     format=plsc.PackFormat.INTERLEAVED)
                hs.append(p1 * q1 + p2 * q2)
            for dist in (1, 2, 4, 8):
                perm = jnp.bitwise_xor(iota, dist)
                lmask = (iota & dist) == 0
                nxt = []
                for m in range(0, len(hs), 2):
                    pa = hs[m] + hs[m].at[perm].get(mode="promise_in_bounds")
                    pb_ = hs[m + 1] + hs[m + 1].at[perm].get(
                        mode="promise_in_bounds")
                    nxt.append(jnp.where(lmask, pa, pb_))
                hs = nxt
            acc = hs[0]
            pb = pb_v[pl.ds(g * GRP, GRP)]
            qb = qb_v[pl.ds(g * GRP, GRP)]
            idrow = cc * SPC + (g // 8)
            idoff = (g % 8) * GRP
            uu = uid_v[idrow, pl.ds(idoff, GRP)]
            ss = sid_v[idrow, pl.ds(idoff, GRP)]
            pmask = uu != 0
            qmask = ss != 0
            x = (jnp.where(pmask & qmask, acc, 0.0)
                 + jnp.where(pmask, pb, 0.0)
                 + jnp.where(qmask, qb, 0.0))
            o_v[g // 8, pl.ds(idoff, GRP)] = 1.0 / (1.0 + jnp.exp(-x))
            return carry

        lax.fori_loop(0, CH // GRP, grp_body, 0)

    fire(0, 0)
    fire(1, 1)

    @pl.loop(0, nch, step=2)
    def _loop(c):
        for k in range(2):
            cc = c + k
            _, _, _, _, o_v, _, osem = bufs[k]
            drain_gathers(cc, k)

            @pl.when(cc >= 2)
            def _():
                drain_out(k)

            compute(cc, k)
            pltpu.async_copy(
                o_v,
                out.at[pl.ds(cc * SPC, SPC), pl.ds(wid * SUB, SUB)],
                osem)

            @pl.when(cc + 2 < nch)
            def _():
                fire(cc + 2, k)

    drain_out(0)
    drain_out(1)


def kernel(user_id_sequence, skill_id_sequence, P, Q, P_bias, Q_bias):
    B, L = user_id_sequence.shape
    ut = user_id_sequence.T.astype(jnp.int32)
    st = skill_id_sequence.T.astype(jnp.int32)
    mesh = plsc.VectorSubcoreMesh(core_axis_name="c", subcore_axis_name="s")
    dbuf = []
    for _ in range(2):
        dbuf += [
            pltpu.VMEM((CH, D), jnp.bfloat16),
            pltpu.VMEM((CH, D), jnp.bfloat16),
            pltpu.VMEM((CH,), jnp.float32),
            pltpu.VMEM((CH,), jnp.float32),
            pltpu.VMEM((SPC, SUB), jnp.float32),
        ]
    kfn = pl.kernel(
        _mf_body,
        out_type=jax.ShapeDtypeStruct((L, B), jnp.float32),
        mesh=mesh,
        compiler_params=pltpu.CompilerParams(
            needs_layout_passes=False, use_tc_tiling_on_sc=False),
        scratch_types=[
            pltpu.VMEM((L, SUB), jnp.int32),
            pltpu.VMEM((L, SUB), jnp.int32),
        ] + dbuf + [
            pltpu.SemaphoreType.DMA,
            pltpu.SemaphoreType.DMA,
            pltpu.SemaphoreType.DMA,
            pltpu.SemaphoreType.DMA,
        ],
    )
    out2d = kfn(ut, st, P.astype(jnp.bfloat16), Q.astype(jnp.bfloat16), P_bias.reshape(-1), Q_bias.reshape(-1))
    return out2d.T[:, :, None]


# speedup vs baseline: 1.3380x; 1.3380x over previous
import jax
import jax.numpy as jnp
from jax import lax
from jax.experimental import pallas as pl
from jax.experimental.pallas import tpu as pltpu
from jax.experimental.pallas import tpu_sc as plsc

D = 32
NC = 2
NS = 16
NW = NC * NS
SUB = 128
SPC = 4
CH = SUB * SPC
GRP = 16


def _mf_body(ut, st, P, Q, Pb, Qb, out,
             uid_v, sid_v,
             p0, q0, pb0, qb0, o0,
             p1, q1, pb1, qb1, o1,
             g0, g1, os0, os1):
    L = ut.shape[0]
    nch = L // SPC
    wid = lax.axis_index("s") * NC + lax.axis_index("c")
    iota = lax.iota(jnp.int32, GRP)
    bufs = ((p0, q0, pb0, qb0, o0, g0, os0),
            (p1, q1, pb1, qb1, o1, g1, os1))

    pltpu.sync_copy(ut.at[pl.ds(0, L), pl.ds(wid * SUB, SUB)], uid_v)
    pltpu.sync_copy(st.at[pl.ds(0, L), pl.ds(wid * SUB, SUB)], sid_v)

    def fire(cc, k):
        p_v, q_v, pb_v, qb_v, _, gsem, _ = bufs[k]
        for j in range(SPC):
            row = cc * SPC + j
            dst = pl.ds(j * SUB, SUB)
            pltpu.async_copy(P.at[uid_v.at[row]], p_v.at[dst], gsem)
            pltpu.async_copy(Q.at[sid_v.at[row]], q_v.at[dst], gsem)
            pltpu.async_copy(Pb.at[uid_v.at[row]], pb_v.at[dst], gsem)
            pltpu.async_copy(Qb.at[sid_v.at[row]], qb_v.at[dst], gsem)

    def drain_gathers(cc, k):
        p_v, q_v, pb_v, qb_v, _, gsem, _ = bufs[k]
        for j in range(SPC):
            row = cc * SPC + j
            dst = pl.ds(j * SUB, SUB)
            pltpu.make_async_copy(P.at[uid_v.at[row]], p_v.at[dst], gsem).wait()
            pltpu.make_async_copy(Q.at[sid_v.at[row]], q_v.at[dst], gsem).wait()
            pltpu.make_async_copy(Pb.at[uid_v.at[row]], pb_v.at[dst],
                                  gsem).wait()
            pltpu.make_async_copy(Qb.at[sid_v.at[row]], qb_v.at[dst],
                                  gsem).wait()

    def drain_out(k):
        _, _, _, _, o_v, _, osem = bufs[k]
        pltpu.make_async_copy(
            o_v, out.at[pl.ds(0, SPC), pl.ds(0, SUB)], osem).wait()

    def compute(cc, k):
        p_v, q_v, pb_v, qb_v, o_v, _, _ = bufs[k]

        def grp_body(g, carry):
            hs = []
            for r in range(GRP):
                i = g * GRP + r
                p1 = p_v[i, pl.ds(0, GRP)]
                p2 = p_v[i, pl.ds(GRP, GRP)]
                q1 = q_v[i, pl.ds(0, GRP)]
                q2 = q_v[i, pl.ds(GRP, GRP)]
                hs.append(p1 * q1 + p2 * q2)
            for dist in (1, 2, 4, 8):
                perm = jnp.bitwise_xor(iota, dist)
                lmask = (iota & dist) == 0
                nxt = []
                for m in range(0, len(hs), 2):
                    pa = hs[m] + hs[m].at[perm].get(mode="promise_in_bounds")
                    pb_ = hs[m + 1] + hs[m + 1].at[perm].get(
                        mode="promise_in_bounds")
                    nxt.append(jnp.where(lmask, pa, pb_))
                hs = nxt
            acc = hs[0]
            pb = pb_v[pl.ds(g * GRP, GRP)]
            qb = qb_v[pl.ds(g * GRP, GRP)]
            idrow = cc * SPC + (g // 8)
            idoff = (g % 8) * GRP
            uu = uid_v[idrow, pl.ds(idoff, GRP)]
            ss = sid_v[idrow, pl.ds(idoff, GRP)]
            pmask = uu != 0
            qmask = ss != 0
            x = (jnp.where(pmask & qmask, acc, 0.0)
                 + jnp.where(pmask, pb, 0.0)
                 + jnp.where(qmask, qb, 0.0))
            o_v[g // 8, pl.ds(idoff, GRP)] = 1.0 / (1.0 + jnp.exp(-x))
            return carry

        lax.fori_loop(0, CH // GRP, grp_body, 0)

    fire(0, 0)
    fire(1, 1)

    @pl.loop(0, nch, step=2)
    def _loop(c):
        for k in range(2):
            cc = c + k
            _, _, _, _, o_v, _, osem = bufs[k]
            drain_gathers(cc, k)

            @pl.when(cc >= 2)
            def _():
                drain_out(k)

            compute(cc, k)
            pltpu.async_copy(
                o_v,
                out.at[pl.ds(cc * SPC, SPC), pl.ds(wid * SUB, SUB)],
                osem)

            @pl.when(cc + 2 < nch)
            def _():
                fire(cc + 2, k)

    drain_out(0)
    drain_out(1)


def kernel(user_id_sequence, skill_id_sequence, P, Q, P_bias, Q_bias):
    B, L = user_id_sequence.shape
    ut = user_id_sequence.T.astype(jnp.int32)
    st = skill_id_sequence.T.astype(jnp.int32)
    mesh = plsc.VectorSubcoreMesh(core_axis_name="c", subcore_axis_name="s")
    dbuf = []
    for _ in range(2):
        dbuf += [
            pltpu.VMEM((CH, D), jnp.float32),
            pltpu.VMEM((CH, D), jnp.float32),
            pltpu.VMEM((CH,), jnp.float32),
            pltpu.VMEM((CH,), jnp.float32),
            pltpu.VMEM((SPC, SUB), jnp.float32),
        ]
    kfn = pl.kernel(
        _mf_body,
        out_type=jax.ShapeDtypeStruct((L, B), jnp.float32),
        mesh=mesh,
        compiler_params=pltpu.CompilerParams(
            needs_layout_passes=False, use_tc_tiling_on_sc=False),
        scratch_types=[
            pltpu.VMEM((L, SUB), jnp.int32),
            pltpu.VMEM((L, SUB), jnp.int32),
        ] + dbuf + [
            pltpu.SemaphoreType.DMA,
            pltpu.SemaphoreType.DMA,
            pltpu.SemaphoreType.DMA,
            pltpu.SemaphoreType.DMA,
        ],
    )
    out2d = kfn(ut, st, P, Q, P_bias.reshape(-1), Q_bias.reshape(-1))
    return out2d.T[:, :, None]
